# baseline (speedup 1.0000x reference)
import functools

import jax
import jax.numpy as jnp
from jax.experimental import pallas as pl
from jax.experimental.pallas import tpu as pltpu

T, D, E, F, K = 2048, 1024, 64, 256, 2
TR = 32
R = 6144
NB = R // TR


def _ffn_body(be_ref, nu_ref, xs_ref, gw_ref, uw_ref, dw_ref, ys_ref):
    i = pl.program_id(0)

    @pl.when(i < nu_ref[0])
    def _():
        xb = xs_ref[...]
        gw = gw_ref[0]
        uw = uw_ref[0]
        dw = dw_ref[0]
        cdims = (((1,), (1,)), ((), ()))
        g = jax.lax.dot_general(xb, gw, cdims,
                                preferred_element_type=jnp.float32)
        u = jax.lax.dot_general(xb, uw, cdims,
                                preferred_element_type=jnp.float32)
        h = (g * jax.nn.sigmoid(g)) * u
        ys_ref[...] = jax.lax.dot_general(h, dw, cdims,
                                          preferred_element_type=jnp.float32)


def _grouped_ffn(block_expert, num_used, xs, gpw, upw, dpw):
    grid_spec = pltpu.PrefetchScalarGridSpec(
        num_scalar_prefetch=2,
        grid=(NB,),
        in_specs=[
            pl.BlockSpec((TR, D), lambda i, be, nu: (i, 0)),
            pl.BlockSpec((1, F, D), lambda i, be, nu: (be[i], 0, 0)),
            pl.BlockSpec((1, F, D), lambda i, be, nu: (be[i], 0, 0)),
            pl.BlockSpec((1, D, F), lambda i, be, nu: (be[i], 0, 0)),
        ],
        out_specs=pl.BlockSpec((TR, D), lambda i, be, nu: (i, 0)),
    )
    return pl.pallas_call(
        _ffn_body,
        grid_spec=grid_spec,
        out_shape=jax.ShapeDtypeStruct((R, D), jnp.float32),
    )(block_expert, num_used, xs, gpw, upw, dpw)


def _top2_exact(logits):
    idx1 = jnp.argmax(logits, axis=-1).astype(jnp.int32)
    v1 = jnp.take_along_axis(logits, idx1[:, None], axis=1)[:, 0]
    masked = jnp.where(jax.nn.one_hot(idx1, E, dtype=jnp.bool_),
                       -jnp.inf, logits)
    idx2 = jnp.argmax(masked, axis=-1).astype(jnp.int32)
    v2 = jnp.take_along_axis(logits, idx2[:, None], axis=1)[:, 0]
    return jnp.stack([v1, v2], axis=1), jnp.stack([idx1, idx2], axis=1)


def kernel(hidden_states, gate_w, gate_proj_w, up_proj_w, down_proj_w):
    b, s, d = hidden_states.shape
    x = hidden_states.reshape(-1, d)

    logits = x @ gate_w.T
    topk_w_raw, topk_idx = _top2_exact(logits)
    tw = jax.nn.softmax(topk_w_raw, axis=-1)
    tw = tw / (tw.sum(axis=-1, keepdims=True) + 1e-20)

    e_flat = topk_idx.reshape(-1)
    oh = jax.nn.one_hot(e_flat, E, dtype=jnp.int32)
    counts = oh.sum(axis=0)
    rank = jnp.take_along_axis(jnp.cumsum(oh, axis=0) - oh,
                               e_flat[:, None], axis=1)[:, 0]
    blocks_per_e = (counts + TR - 1) // TR
    pad_off = jnp.concatenate(
        [jnp.zeros((1,), jnp.int32),
         jnp.cumsum(blocks_per_e * TR).astype(jnp.int32)])
    slot = pad_off[e_flat] + rank
    tid = jnp.arange(T * K, dtype=jnp.int32) // K
    tok_for_slot = jnp.zeros((R,), jnp.int32).at[slot].set(tid)
    xs = x[tok_for_slot]
    block_expert = jnp.clip(
        jnp.searchsorted(pad_off, jnp.arange(NB, dtype=jnp.int32) * TR,
                         side='right') - 1,
        0, E - 1).astype(jnp.int32)
    num_used = (pad_off[E:E + 1] // TR).astype(jnp.int32)

    ys = _grouped_ffn(block_expert, num_used, xs,
                      gate_proj_w, up_proj_w, down_proj_w)

    slot2 = slot.reshape(T, K)
    out = tw[:, 0:1] * ys[slot2[:, 0]] + tw[:, 1:2] * ys[slot2[:, 1]]
    return out.reshape(b, s, d)


# speedup vs baseline: 1.1336x; 1.1336x over previous
import functools

import jax
import jax.numpy as jnp
from jax.experimental import pallas as pl
from jax.experimental.pallas import tpu as pltpu

T, D, E, F, K = 2048, 1024, 64, 256, 2
TR = 64
R = 8192
NB = R // TR


def _ffn_body(be_ref, nu_ref, xs_ref, gw_ref, uw_ref, dw_ref, ys_ref):
    i = pl.program_id(0)

    @pl.when(i < nu_ref[0])
    def _():
        xb = xs_ref[...]
        gw = gw_ref[0]
        uw = uw_ref[0]
        dw = dw_ref[0]
        cdims = (((1,), (1,)), ((), ()))
        g = jax.lax.dot_general(xb, gw, cdims,
                                preferred_element_type=jnp.float32)
        u = jax.lax.dot_general(xb, uw, cdims,
                                preferred_element_type=jnp.float32)
        h = (g * jax.nn.sigmoid(g)) * u
        ys_ref[...] = jax.lax.dot_general(h, dw, cdims,
                                          preferred_element_type=jnp.float32)


def _grouped_ffn(block_expert, num_used, xs, gpw, upw, dpw):
    grid_spec = pltpu.PrefetchScalarGridSpec(
        num_scalar_prefetch=2,
        grid=(NB,),
        in_specs=[
            pl.BlockSpec((TR, D), lambda i, be, nu: (i, 0)),
            pl.BlockSpec((1, F, D), lambda i, be, nu: (be[i], 0, 0)),
            pl.BlockSpec((1, F, D), lambda i, be, nu: (be[i], 0, 0)),
            pl.BlockSpec((1, D, F), lambda i, be, nu: (be[i], 0, 0)),
        ],
        out_specs=pl.BlockSpec((TR, D), lambda i, be, nu: (i, 0)),
    )
    return pl.pallas_call(
        _ffn_body,
        grid_spec=grid_spec,
        out_shape=jax.ShapeDtypeStruct((R, D), jnp.float32),
    )(block_expert, num_used, xs, gpw, upw, dpw)


def _top2_exact(logits):
    idx1 = jnp.argmax(logits, axis=-1).astype(jnp.int32)
    v1 = jnp.take_along_axis(logits, idx1[:, None], axis=1)[:, 0]
    masked = jnp.where(jax.nn.one_hot(idx1, E, dtype=jnp.bool_),
                       -jnp.inf, logits)
    idx2 = jnp.argmax(masked, axis=-1).astype(jnp.int32)
    v2 = jnp.take_along_axis(logits, idx2[:, None], axis=1)[:, 0]
    return jnp.stack([v1, v2], axis=1), jnp.stack([idx1, idx2], axis=1)


def kernel(hidden_states, gate_w, gate_proj_w, up_proj_w, down_proj_w):
    b, s, d = hidden_states.shape
    x = hidden_states.reshape(-1, d)

    logits = x @ gate_w.T
    topk_w_raw, topk_idx = _top2_exact(logits)
    tw = jax.nn.softmax(topk_w_raw, axis=-1)
    tw = tw / (tw.sum(axis=-1, keepdims=True) + 1e-20)

    e_flat = topk_idx.reshape(-1)
    oh = jax.nn.one_hot(e_flat, E, dtype=jnp.int16)
    counts = oh.sum(axis=0, dtype=jnp.int32)
    rank = jnp.take_along_axis(
        (jnp.cumsum(oh, axis=0) - oh).astype(jnp.int32),
        e_flat[:, None], axis=1)[:, 0]
    blocks_per_e = (counts + TR - 1) // TR
    pad_off = jnp.concatenate(
        [jnp.zeros((1,), jnp.int32),
         jnp.cumsum(blocks_per_e * TR).astype(jnp.int32)])
    slot = pad_off[e_flat] + rank
    tid = jnp.arange(T * K, dtype=jnp.int32) // K
    tok_for_slot = jnp.zeros((R,), jnp.int32).at[slot].set(tid)
    xs = x[tok_for_slot]
    block_expert = jnp.clip(
        jnp.searchsorted(pad_off, jnp.arange(NB, dtype=jnp.int32) * TR,
                         side='right') - 1,
        0, E - 1).astype(jnp.int32)
    num_used = (pad_off[E:E + 1] // TR).astype(jnp.int32)

    ys = _grouped_ffn(block_expert, num_used, xs,
                      gate_proj_w, up_proj_w, down_proj_w)

    slot2 = slot.reshape(T, K)
    out = tw[:, 0:1] * ys[slot2[:, 0]] + tw[:, 1:2] * ys[slot2[:, 1]]
    return out.reshape(b, s, d)


# speedup vs baseline: 1.1921x; 1.0515x over previous
import functools

import jax
import jax.numpy as jnp
from jax.experimental import pallas as pl
from jax.experimental.pallas import tpu as pltpu

T, D, E, F, K = 2048, 1024, 64, 256, 2
TR = 64
R = 8192
NB = R // TR


def _ffn_body(be_ref, nu_ref, xs_ref, gw_ref, uw_ref, dw_ref, ys_ref):
    i = pl.program_id(0)

    @pl.when(i < nu_ref[0])
    def _():
        xb = xs_ref[...]
        gw = gw_ref[0]
        uw = uw_ref[0]
        dw = dw_ref[0]
        cdims = (((1,), (1,)), ((), ()))
        g = jax.lax.dot_general(xb, gw, cdims,
                                preferred_element_type=jnp.float32)
        u = jax.lax.dot_general(xb, uw, cdims,
                                preferred_element_type=jnp.float32)
        h = (g * jax.nn.sigmoid(g)) * u
        ys_ref[...] = jax.lax.dot_general(h, dw, cdims,
                                          preferred_element_type=jnp.float32)


def _grouped_ffn(block_expert, num_used, xs, gpw, upw, dpw):
    grid_spec = pltpu.PrefetchScalarGridSpec(
        num_scalar_prefetch=2,
        grid=(NB,),
        in_specs=[
            pl.BlockSpec((TR, D),
                         lambda i, be, nu: (jnp.minimum(i, nu[0] - 1), 0)),
            pl.BlockSpec((1, F, D), lambda i, be, nu: (be[i], 0, 0)),
            pl.BlockSpec((1, F, D), lambda i, be, nu: (be[i], 0, 0)),
            pl.BlockSpec((1, D, F), lambda i, be, nu: (be[i], 0, 0)),
        ],
        out_specs=pl.BlockSpec(
            (TR, D), lambda i, be, nu: (jnp.minimum(i, nu[0] - 1), 0)),
    )
    return pl.pallas_call(
        _ffn_body,
        grid_spec=grid_spec,
        out_shape=jax.ShapeDtypeStruct((R, D), jnp.float32),
    )(block_expert, num_used, xs, gpw, upw, dpw)


def _top2_exact(logits):
    idx1 = jnp.argmax(logits, axis=-1).astype(jnp.int32)
    v1 = jnp.take_along_axis(logits, idx1[:, None], axis=1)[:, 0]
    masked = jnp.where(jax.nn.one_hot(idx1, E, dtype=jnp.bool_),
                       -jnp.inf, logits)
    idx2 = jnp.argmax(masked, axis=-1).astype(jnp.int32)
    v2 = jnp.take_along_axis(logits, idx2[:, None], axis=1)[:, 0]
    return jnp.stack([v1, v2], axis=1), jnp.stack([idx1, idx2], axis=1)


def kernel(hidden_states, gate_w, gate_proj_w, up_proj_w, down_proj_w):
    b, s, d = hidden_states.shape
    x = hidden_states.reshape(-1, d)

    logits = x @ gate_w.T
    topk_w_raw, topk_idx = _top2_exact(logits)
    tw = jax.nn.softmax(topk_w_raw, axis=-1)
    tw = tw / (tw.sum(axis=-1, keepdims=True) + 1e-20)

    e_flat = topk_idx.reshape(-1)
    oh = jax.nn.one_hot(e_flat, E, dtype=jnp.int16)
    counts = oh.sum(axis=0, dtype=jnp.int32)
    rank = jnp.take_along_axis(
        (jnp.cumsum(oh, axis=0) - oh).astype(jnp.int32),
        e_flat[:, None], axis=1)[:, 0]
    blocks_per_e = (counts + TR - 1) // TR
    pad_off = jnp.concatenate(
        [jnp.zeros((1,), jnp.int32),
         jnp.cumsum(blocks_per_e * TR).astype(jnp.int32)])
    slot = pad_off[e_flat] + rank
    tid = jnp.arange(T * K, dtype=jnp.int32) // K
    tok_for_slot = jnp.zeros((R,), jnp.int32).at[slot].set(tid)
    xs = x[tok_for_slot]
    num_used = (pad_off[E:E + 1] // TR).astype(jnp.int32)
    blk = jnp.minimum(jnp.arange(NB, dtype=jnp.int32), num_used[0] - 1)
    block_expert = jnp.clip(
        jnp.searchsorted(pad_off, blk * TR, side='right') - 1,
        0, E - 1).astype(jnp.int32)

    ys = _grouped_ffn(block_expert, num_used, xs,
                      gate_proj_w, up_proj_w, down_proj_w)

    slot2 = slot.reshape(T, K)
    out = tw[:, 0:1] * ys[slot2[:, 0]] + tw[:, 1:2] * ys[slot2[:, 1]]
    return out.reshape(b, s, d)


# speedup vs baseline: 1.3602x; 1.1411x over previous
import functools

import jax
import jax.numpy as jnp
from jax import lax
from jax.experimental import pallas as pl
from jax.experimental.pallas import tpu as pltpu
from jax.experimental.pallas import tpu_sc as plsc

T, D, E, F, K = 2048, 1024, 64, 256, 2
TR = 64
R = 8192
NB = R // TR

_NC, _NS = 2, 16
NW = _NC * _NS
TPW = T // NW


def _sc_mesh():
    return plsc.VectorSubcoreMesh(core_axis_name="c", subcore_axis_name="s")


def _wid():
    return lax.axis_index("s") * _NC + lax.axis_index("c")


def _dispatch_scatter(x, se, so):
    @functools.partial(
        pl.kernel,
        out_type=jax.ShapeDtypeStruct((R, D), jnp.float32),
        mesh=_sc_mesh(),
        scratch_types=[
            pltpu.VMEM((TPW, D), jnp.float32),
            pltpu.VMEM((TPW,), jnp.int32),
            pltpu.VMEM((TPW,), jnp.int32),
            pltpu.SemaphoreType.DMA,
        ],
    )
    def k(x_hbm, se_hbm, so_hbm, xs_hbm, rows_v, se_v, so_v, sem):
        w = _wid()
        base = w * TPW
        pltpu.sync_copy(se_hbm.at[pl.ds(base, TPW)], se_v)
        pltpu.sync_copy(so_hbm.at[pl.ds(base, TPW)], so_v)
        pltpu.sync_copy(x_hbm.at[pl.ds(base, TPW)], rows_v)
        c1 = pltpu.async_copy(rows_v, xs_hbm.at[se_v], sem)
        c2 = pltpu.async_copy(rows_v, xs_hbm.at[so_v], sem)
        c1.wait()
        c2.wait()

    return k(x, se, so)


_CH = 32


def _combine(ys, se, so, tw0, tw1):
    @functools.partial(
        pl.kernel,
        out_type=jax.ShapeDtypeStruct((T, D), jnp.float32),
        mesh=_sc_mesh(),
        scratch_types=[
            pltpu.VMEM((_CH, D), jnp.float32),
            pltpu.VMEM((_CH, D), jnp.float32),
            pltpu.VMEM((_CH, D), jnp.float32),
            pltpu.VMEM((_CH,), jnp.int32),
            pltpu.VMEM((_CH,), jnp.int32),
            pltpu.VMEM((_CH, 16), jnp.float32),
            pltpu.VMEM((_CH, 16), jnp.float32),
            pltpu.SemaphoreType.DMA,
        ],
    )
    def k(ys_hbm, se_hbm, so_hbm, tw0_hbm, tw1_hbm, out_hbm,
          r0_v, r1_v, ob_v, s0_v, s1_v, w0_v, w1_v, sem):
        w = _wid()
        for c in range(TPW // _CH):
            base = w * TPW + c * _CH
            pltpu.sync_copy(se_hbm.at[pl.ds(base, _CH)], s0_v)
            pltpu.sync_copy(so_hbm.at[pl.ds(base, _CH)], s1_v)
            pltpu.sync_copy(tw0_hbm.at[pl.ds(base, _CH)], w0_v)
            pltpu.sync_copy(tw1_hbm.at[pl.ds(base, _CH)], w1_v)
            g0 = pltpu.async_copy(ys_hbm.at[s0_v], r0_v, sem)
            g1 = pltpu.async_copy(ys_hbm.at[s1_v], r1_v, sem)
            g0.wait()
            g1.wait()
            for t in range(_CH):
                w0 = w0_v[t, :]
                w1 = w1_v[t, :]

                def body(j, _):
                    sl = pl.ds(j * 16, 16)
                    ob_v[t, sl] = w0 * r0_v[t, sl] + w1 * r1_v[t, sl]
                    return 0

                lax.fori_loop(0, D // 16, body, 0, unroll=8)
            pltpu.sync_copy(ob_v, out_hbm.at[pl.ds(base, _CH)])

    return k(ys, se, so, tw0, tw1)


def _ffn_body(be_ref, nu_ref, xs_ref, gw_ref, uw_ref, dw_ref, ys_ref):
    i = pl.program_id(0)

    @pl.when(i < nu_ref[0])
    def _():
        xb = xs_ref[...]
        gw = gw_ref[0]
        uw = uw_ref[0]
        dw = dw_ref[0]
        cdims = (((1,), (1,)), ((), ()))
        g = jax.lax.dot_general(xb, gw, cdims,
                                preferred_element_type=jnp.float32)
        u = jax.lax.dot_general(xb, uw, cdims,
                                preferred_element_type=jnp.float32)
        h = (g * jax.nn.sigmoid(g)) * u
        ys_ref[...] = jax.lax.dot_general(h, dw, cdims,
                                          preferred_element_type=jnp.float32)


def _grouped_ffn(block_expert, num_used, xs, gpw, upw, dpw):
    grid_spec = pltpu.PrefetchScalarGridSpec(
        num_scalar_prefetch=2,
        grid=(NB,),
        in_specs=[
            pl.BlockSpec((TR, D),
                         lambda i, be, nu: (jnp.minimum(i, nu[0] - 1), 0)),
            pl.BlockSpec((1, F, D), lambda i, be, nu: (be[i], 0, 0)),
            pl.BlockSpec((1, F, D), lambda i, be, nu: (be[i], 0, 0)),
            pl.BlockSpec((1, D, F), lambda i, be, nu: (be[i], 0, 0)),
        ],
        out_specs=pl.BlockSpec(
            (TR, D), lambda i, be, nu: (jnp.minimum(i, nu[0] - 1), 0)),
    )
    return pl.pallas_call(
        _ffn_body,
        grid_spec=grid_spec,
        out_shape=jax.ShapeDtypeStruct((R, D), jnp.float32),
    )(block_expert, num_used, xs, gpw, upw, dpw)


def _top2_exact(logits):
    idx1 = jnp.argmax(logits, axis=-1).astype(jnp.int32)
    v1 = jnp.take_along_axis(logits, idx1[:, None], axis=1)[:, 0]
    masked = jnp.where(jax.nn.one_hot(idx1, E, dtype=jnp.bool_),
                       -jnp.inf, logits)
    idx2 = jnp.argmax(masked, axis=-1).astype(jnp.int32)
    v2 = jnp.take_along_axis(logits, idx2[:, None], axis=1)[:, 0]
    return jnp.stack([v1, v2], axis=1), jnp.stack([idx1, idx2], axis=1)


def kernel(hidden_states, gate_w, gate_proj_w, up_proj_w, down_proj_w):
    b, s, d = hidden_states.shape
    x = hidden_states.reshape(-1, d)

    logits = x @ gate_w.T
    topk_w_raw, topk_idx = _top2_exact(logits)
    tw = jax.nn.softmax(topk_w_raw, axis=-1)
    tw = tw / (tw.sum(axis=-1, keepdims=True) + 1e-20)

    e_flat = topk_idx.reshape(-1)
    oh = jax.nn.one_hot(e_flat, E, dtype=jnp.int16)
    counts = oh.sum(axis=0, dtype=jnp.int32)
    rank = jnp.take_along_axis(
        (jnp.cumsum(oh, axis=0) - oh).astype(jnp.int32),
        e_flat[:, None], axis=1)[:, 0]
    blocks_per_e = (counts + TR - 1) // TR
    pad_off = jnp.concatenate(
        [jnp.zeros((1,), jnp.int32),
         jnp.cumsum(blocks_per_e * TR).astype(jnp.int32)])
    slot = pad_off[e_flat] + rank
    slot2 = slot.reshape(T, K)
    se = slot2[:, 0]
    so = slot2[:, 1]
    xs = _dispatch_scatter(x, se, so)
    num_used = (pad_off[E:E + 1] // TR).astype(jnp.int32)
    blk = jnp.minimum(jnp.arange(NB, dtype=jnp.int32), num_used[0] - 1)
    block_expert = jnp.clip(
        jnp.searchsorted(pad_off, blk * TR, side='right') - 1,
        0, E - 1).astype(jnp.int32)

    ys = _grouped_ffn(block_expert, num_used, xs,
                      gate_proj_w, up_proj_w, down_proj_w)

    tw0b = jnp.broadcast_to(tw[:, 0:1], (T, 16))
    tw1b = jnp.broadcast_to(tw[:, 1:2], (T, 16))
    out = _combine(ys, se, so, tw0b, tw1b)
    return out.reshape(b, s, d)


# speedup vs baseline: 1.3629x; 1.0019x over previous
import functools

import jax
import jax.numpy as jnp
from jax import lax
from jax.experimental import pallas as pl
from jax.experimental.pallas import tpu as pltpu
from jax.experimental.pallas import tpu_sc as plsc

T, D, E, F, K = 2048, 1024, 64, 256, 2
TR = 64
R = 8192
NB = R // TR

_NC, _NS = 2, 16
NW = _NC * _NS
TPW = T // NW


def _sc_mesh():
    return plsc.VectorSubcoreMesh(core_axis_name="c", subcore_axis_name="s")


def _wid():
    return lax.axis_index("s") * _NC + lax.axis_index("c")


def _dispatch_scatter(x, se, so):
    @functools.partial(
        pl.kernel,
        out_type=jax.ShapeDtypeStruct((R, D), jnp.float32),
        mesh=_sc_mesh(),
        scratch_types=[
            pltpu.VMEM((TPW, D), jnp.float32),
            pltpu.VMEM((TPW,), jnp.int32),
            pltpu.VMEM((TPW,), jnp.int32),
            pltpu.SemaphoreType.DMA,
        ],
    )
    def k(x_hbm, se_hbm, so_hbm, xs_hbm, rows_v, se_v, so_v, sem):
        w = _wid()
        base = w * TPW
        pltpu.sync_copy(se_hbm.at[pl.ds(base, TPW)], se_v)
        pltpu.sync_copy(so_hbm.at[pl.ds(base, TPW)], so_v)
        pltpu.sync_copy(x_hbm.at[pl.ds(base, TPW)], rows_v)
        c1 = pltpu.async_copy(rows_v, xs_hbm.at[se_v], sem)
        c2 = pltpu.async_copy(rows_v, xs_hbm.at[so_v], sem)
        c1.wait()
        c2.wait()

    return k(x, se, so)


_CH = 16
_NCH = TPW // _CH


def _combine(ys, se, so, tw0, tw1):
    @functools.partial(
        pl.kernel,
        out_type=jax.ShapeDtypeStruct((T, D), jnp.float32),
        mesh=_sc_mesh(),
        scratch_types=[
            pltpu.VMEM((2, _CH, D), jnp.float32),
            pltpu.VMEM((2, _CH, D), jnp.float32),
            pltpu.VMEM((_CH, D), jnp.float32),
            pltpu.VMEM((2, _CH), jnp.int32),
            pltpu.VMEM((2, _CH), jnp.int32),
            pltpu.VMEM((_CH, 16), jnp.float32),
            pltpu.VMEM((_CH, 16), jnp.float32),
            pltpu.SemaphoreType.DMA,
            pltpu.SemaphoreType.DMA,
        ],
    )
    def k(ys_hbm, se_hbm, so_hbm, tw0_hbm, tw1_hbm, out_hbm,
          r0_v, r1_v, ob_v, s0_v, s1_v, w0_v, w1_v, sem0, sem1):
        w = _wid()
        sems = (sem0, sem1)

        def fire(c):
            p = c % 2
            base = w * TPW + c * _CH
            pltpu.sync_copy(se_hbm.at[pl.ds(base, _CH)], s0_v.at[p])
            pltpu.sync_copy(so_hbm.at[pl.ds(base, _CH)], s1_v.at[p])
            g0 = pltpu.async_copy(ys_hbm.at[s0_v.at[p]], r0_v.at[p], sems[p])
            g1 = pltpu.async_copy(ys_hbm.at[s1_v.at[p]], r1_v.at[p], sems[p])
            return g0, g1

        hs = fire(0)
        for c in range(_NCH):
            nxt = fire(c + 1) if c + 1 < _NCH else None
            base = w * TPW + c * _CH
            pltpu.sync_copy(tw0_hbm.at[pl.ds(base, _CH)], w0_v)
            pltpu.sync_copy(tw1_hbm.at[pl.ds(base, _CH)], w1_v)
            hs[0].wait()
            hs[1].wait()
            p = c % 2
            for t in range(_CH):
                w0 = w0_v[t, :]
                w1 = w1_v[t, :]

                def body(j, _):
                    sl = pl.ds(j * 16, 16)
                    ob_v[t, sl] = (w0 * r0_v[p, t, sl]
                                   + w1 * r1_v[p, t, sl])
                    return 0

                lax.fori_loop(0, D // 16, body, 0, unroll=8)
            pltpu.sync_copy(ob_v, out_hbm.at[pl.ds(base, _CH)])
            hs = nxt

    return k(ys, se, so, tw0, tw1)


def _ffn_body(be_ref, nu_ref, xs_ref, gw_ref, uw_ref, dw_ref, ys_ref):
    i = pl.program_id(0)

    @pl.when(i < nu_ref[0])
    def _():
        xb = xs_ref[...]
        gw = gw_ref[0]
        uw = uw_ref[0]
        dw = dw_ref[0]
        cdims = (((1,), (1,)), ((), ()))
        g = jax.lax.dot_general(xb, gw, cdims,
                                preferred_element_type=jnp.float32)
        u = jax.lax.dot_general(xb, uw, cdims,
                                preferred_element_type=jnp.float32)
        h = (g * jax.nn.sigmoid(g)) * u
        ys_ref[...] = jax.lax.dot_general(h, dw, cdims,
                                          preferred_element_type=jnp.float32)


def _grouped_ffn(block_expert, num_used, xs, gpw, upw, dpw):
    grid_spec = pltpu.PrefetchScalarGridSpec(
        num_scalar_prefetch=2,
        grid=(NB,),
        in_specs=[
            pl.BlockSpec((TR, D),
                         lambda i, be, nu: (jnp.minimum(i, nu[0] - 1), 0)),
            pl.BlockSpec((1, F, D), lambda i, be, nu: (be[i], 0, 0)),
            pl.BlockSpec((1, F, D), lambda i, be, nu: (be[i], 0, 0)),
            pl.BlockSpec((1, D, F), lambda i, be, nu: (be[i], 0, 0)),
        ],
        out_specs=pl.BlockSpec(
            (TR, D), lambda i, be, nu: (jnp.minimum(i, nu[0] - 1), 0)),
    )
    return pl.pallas_call(
        _ffn_body,
        grid_spec=grid_spec,
        out_shape=jax.ShapeDtypeStruct((R, D), jnp.float32),
    )(block_expert, num_used, xs, gpw, upw, dpw)


def _top2_exact(logits):
    idx1 = jnp.argmax(logits, axis=-1).astype(jnp.int32)
    v1 = jnp.take_along_axis(logits, idx1[:, None], axis=1)[:, 0]
    masked = jnp.where(jax.nn.one_hot(idx1, E, dtype=jnp.bool_),
                       -jnp.inf, logits)
    idx2 = jnp.argmax(masked, axis=-1).astype(jnp.int32)
    v2 = jnp.take_along_axis(logits, idx2[:, None], axis=1)[:, 0]
    return jnp.stack([v1, v2], axis=1), jnp.stack([idx1, idx2], axis=1)


def kernel(hidden_states, gate_w, gate_proj_w, up_proj_w, down_proj_w):
    b, s, d = hidden_states.shape
    x = hidden_states.reshape(-1, d)

    logits = x @ gate_w.T
    topk_w_raw, topk_idx = _top2_exact(logits)
    tw = jax.nn.softmax(topk_w_raw, axis=-1)
    tw = tw / (tw.sum(axis=-1, keepdims=True) + 1e-20)

    e_flat = topk_idx.reshape(-1)
    oh = jax.nn.one_hot(e_flat, E, dtype=jnp.int16)
    counts = oh.sum(axis=0, dtype=jnp.int32)
    rank = jnp.take_along_axis(
        (jnp.cumsum(oh, axis=0) - oh).astype(jnp.int32),
        e_flat[:, None], axis=1)[:, 0]
    blocks_per_e = (counts + TR - 1) // TR
    pad_off = jnp.concatenate(
        [jnp.zeros((1,), jnp.int32),
         jnp.cumsum(blocks_per_e * TR).astype(jnp.int32)])
    slot = pad_off[e_flat] + rank
    slot2 = slot.reshape(T, K)
    se = slot2[:, 0]
    so = slot2[:, 1]
    xs = _dispatch_scatter(x, se, so)
    num_used = (pad_off[E:E + 1] // TR).astype(jnp.int32)
    blk = jnp.minimum(jnp.arange(NB, dtype=jnp.int32), num_used[0] - 1)
    block_expert = jnp.clip(
        jnp.searchsorted(pad_off, blk * TR, side='right') - 1,
        0, E - 1).astype(jnp.int32)

    ys = _grouped_ffn(block_expert, num_used, xs,
                      gate_proj_w, up_proj_w, down_proj_w)

    tw0b = jnp.broadcast_to(tw[:, 0:1], (T, 16))
    tw1b = jnp.broadcast_to(tw[:, 1:2], (T, 16))
    out = _combine(ys, se, so, tw0b, tw1b)
    return out.reshape(b, s, d)


# speedup vs baseline: 1.3668x; 1.0029x over previous
import functools

import jax
import jax.numpy as jnp
from jax import lax
from jax.experimental import pallas as pl
from jax.experimental.pallas import tpu as pltpu
from jax.experimental.pallas import tpu_sc as plsc

T, D, E, F, K = 2048, 1024, 64, 256, 2
TR = 64
R = 8192
NB = R // TR

_NC, _NS = 2, 16
NW = _NC * _NS
TPW = T // NW


def _sc_mesh():
    return plsc.VectorSubcoreMesh(core_axis_name="c", subcore_axis_name="s")


def _wid():
    return lax.axis_index("s") * _NC + lax.axis_index("c")


_HF = TPW // 2


def _dispatch_scatter(x, se, so):
    @functools.partial(
        pl.kernel,
        out_type=jax.ShapeDtypeStruct((R, D), jnp.float32),
        mesh=_sc_mesh(),
        scratch_types=[
            pltpu.VMEM((TPW, D), jnp.float32),
            pltpu.VMEM((_HF,), jnp.int32),
            pltpu.VMEM((_HF,), jnp.int32),
            pltpu.VMEM((_HF,), jnp.int32),
            pltpu.VMEM((_HF,), jnp.int32),
            pltpu.SemaphoreType.DMA,
        ],
    )
    def k(x_hbm, se_hbm, so_hbm, xs_hbm, rows_v,
          se0_v, so0_v, se1_v, so1_v, sem):
        w = _wid()
        base = w * TPW
        pltpu.sync_copy(se_hbm.at[pl.ds(base, _HF)], se0_v)
        pltpu.sync_copy(so_hbm.at[pl.ds(base, _HF)], so0_v)
        pltpu.sync_copy(x_hbm.at[pl.ds(base, _HF)],
                        rows_v.at[pl.ds(0, _HF)])
        c0 = pltpu.async_copy(rows_v.at[pl.ds(0, _HF)],
                              xs_hbm.at[se0_v], sem)
        c1 = pltpu.async_copy(rows_v.at[pl.ds(0, _HF)],
                              xs_hbm.at[so0_v], sem)
        pltpu.sync_copy(se_hbm.at[pl.ds(base + _HF, _HF)], se1_v)
        pltpu.sync_copy(so_hbm.at[pl.ds(base + _HF, _HF)], so1_v)
        pltpu.sync_copy(x_hbm.at[pl.ds(base + _HF, _HF)],
                        rows_v.at[pl.ds(_HF, _HF)])
        c2 = pltpu.async_copy(rows_v.at[pl.ds(_HF, _HF)],
                              xs_hbm.at[se1_v], sem)
        c3 = pltpu.async_copy(rows_v.at[pl.ds(_HF, _HF)],
                              xs_hbm.at[so1_v], sem)
        c0.wait()
        c1.wait()
        c2.wait()
        c3.wait()

    return k(x, se, so)


_CH = 16
_NCH = TPW // _CH


def _combine(ys, se, so, tw0, tw1):
    @functools.partial(
        pl.kernel,
        out_type=jax.ShapeDtypeStruct((T, D), jnp.float32),
        mesh=_sc_mesh(),
        scratch_types=[
            pltpu.VMEM((2, _CH, D), jnp.float32),
            pltpu.VMEM((2, _CH, D), jnp.float32),
            pltpu.VMEM((_CH, D), jnp.float32),
            pltpu.VMEM((2, _CH), jnp.int32),
            pltpu.VMEM((2, _CH), jnp.int32),
            pltpu.VMEM((_CH, 16), jnp.float32),
            pltpu.VMEM((_CH, 16), jnp.float32),
            pltpu.SemaphoreType.DMA,
            pltpu.SemaphoreType.DMA,
        ],
    )
    def k(ys_hbm, se_hbm, so_hbm, tw0_hbm, tw1_hbm, out_hbm,
          r0_v, r1_v, ob_v, s0_v, s1_v, w0_v, w1_v, sem0, sem1):
        w = _wid()
        sems = (sem0, sem1)

        def fire(c):
            p = c % 2
            base = w * TPW + c * _CH
            pltpu.sync_copy(se_hbm.at[pl.ds(base, _CH)], s0_v.at[p])
            pltpu.sync_copy(so_hbm.at[pl.ds(base, _CH)], s1_v.at[p])
            g0 = pltpu.async_copy(ys_hbm.at[s0_v.at[p]], r0_v.at[p], sems[p])
            g1 = pltpu.async_copy(ys_hbm.at[s1_v.at[p]], r1_v.at[p], sems[p])
            return g0, g1

        hs = fire(0)
        for c in range(_NCH):
            nxt = fire(c + 1) if c + 1 < _NCH else None
            base = w * TPW + c * _CH
            pltpu.sync_copy(tw0_hbm.at[pl.ds(base, _CH)], w0_v)
            pltpu.sync_copy(tw1_hbm.at[pl.ds(base, _CH)], w1_v)
            hs[0].wait()
            hs[1].wait()
            p = c % 2
            for t in range(_CH):
                w0 = w0_v[t, :]
                w1 = w1_v[t, :]

                def body(j, _):
                    sl = pl.ds(j * 16, 16)
                    ob_v[t, sl] = (w0 * r0_v[p, t, sl]
                                   + w1 * r1_v[p, t, sl])
                    return 0

                lax.fori_loop(0, D // 16, body, 0, unroll=8)
            pltpu.sync_copy(ob_v, out_hbm.at[pl.ds(base, _CH)])
            hs = nxt

    return k(ys, se, so, tw0, tw1)


def _ffn_body(be_ref, nu_ref, xs_ref, gw_ref, uw_ref, dw_ref, ys_ref):
    i = pl.program_id(0)

    @pl.when(i < nu_ref[0])
    def _():
        xb = xs_ref[...]
        gw = gw_ref[0]
        uw = uw_ref[0]
        dw = dw_ref[0]
        cdims = (((1,), (1,)), ((), ()))
        g = jax.lax.dot_general(xb, gw, cdims,
                                preferred_element_type=jnp.float32)
        u = jax.lax.dot_general(xb, uw, cdims,
                                preferred_element_type=jnp.float32)
        h = (g * jax.nn.sigmoid(g)) * u
        ys_ref[...] = jax.lax.dot_general(h, dw, cdims,
                                          preferred_element_type=jnp.float32)


def _grouped_ffn(block_expert, num_used, xs, gpw, upw, dpw):
    grid_spec = pltpu.PrefetchScalarGridSpec(
        num_scalar_prefetch=2,
        grid=(NB,),
        in_specs=[
            pl.BlockSpec((TR, D),
                         lambda i, be, nu: (jnp.minimum(i, nu[0] - 1), 0)),
            pl.BlockSpec((1, F, D), lambda i, be, nu: (be[i], 0, 0)),
            pl.BlockSpec((1, F, D), lambda i, be, nu: (be[i], 0, 0)),
            pl.BlockSpec((1, D, F), lambda i, be, nu: (be[i], 0, 0)),
        ],
        out_specs=pl.BlockSpec(
            (TR, D), lambda i, be, nu: (jnp.minimum(i, nu[0] - 1), 0)),
    )
    return pl.pallas_call(
        _ffn_body,
        grid_spec=grid_spec,
        out_shape=jax.ShapeDtypeStruct((R, D), jnp.float32),
    )(block_expert, num_used, xs, gpw, upw, dpw)


def _top2_exact(logits):
    idx1 = jnp.argmax(logits, axis=-1).astype(jnp.int32)
    v1 = jnp.take_along_axis(logits, idx1[:, None], axis=1)[:, 0]
    masked = jnp.where(jax.nn.one_hot(idx1, E, dtype=jnp.bool_),
                       -jnp.inf, logits)
    idx2 = jnp.argmax(masked, axis=-1).astype(jnp.int32)
    v2 = jnp.take_along_axis(logits, idx2[:, None], axis=1)[:, 0]
    return jnp.stack([v1, v2], axis=1), jnp.stack([idx1, idx2], axis=1)


def kernel(hidden_states, gate_w, gate_proj_w, up_proj_w, down_proj_w):
    b, s, d = hidden_states.shape
    x = hidden_states.reshape(-1, d)

    logits = x @ gate_w.T
    topk_w_raw, topk_idx = _top2_exact(logits)
    tw = jax.nn.softmax(topk_w_raw, axis=-1)
    tw = tw / (tw.sum(axis=-1, keepdims=True) + 1e-20)

    e_flat = topk_idx.reshape(-1)
    oh = jax.nn.one_hot(e_flat, E, dtype=jnp.int16)
    counts = oh.sum(axis=0, dtype=jnp.int32)
    rank = jnp.take_along_axis(
        (jnp.cumsum(oh, axis=0) - oh).astype(jnp.int32),
        e_flat[:, None], axis=1)[:, 0]
    blocks_per_e = (counts + TR - 1) // TR
    pad_off = jnp.concatenate(
        [jnp.zeros((1,), jnp.int32),
         jnp.cumsum(blocks_per_e * TR).astype(jnp.int32)])
    slot = pad_off[e_flat] + rank
    slot2 = slot.reshape(T, K)
    se = slot2[:, 0]
    so = slot2[:, 1]
    xs = _dispatch_scatter(x, se, so)
    num_used = (pad_off[E:E + 1] // TR).astype(jnp.int32)
    blk = jnp.minimum(jnp.arange(NB, dtype=jnp.int32), num_used[0] - 1)
    block_expert = jnp.clip(
        jnp.searchsorted(pad_off, blk * TR, side='right') - 1,
        0, E - 1).astype(jnp.int32)

    ys = _grouped_ffn(block_expert, num_used, xs,
                      gate_proj_w, up_proj_w, down_proj_w)

    tw0b = jnp.broadcast_to(tw[:, 0:1], (T, 16))
    tw1b = jnp.broadcast_to(tw[:, 1:2], (T, 16))
    out = _combine(ys, se, so, tw0b, tw1b)
    return out.reshape(b, s, d)


# speedup vs baseline: 1.3982x; 1.0230x over previous
import functools

import jax
import jax.numpy as jnp
from jax import lax
from jax.experimental import pallas as pl
from jax.experimental.pallas import tpu as pltpu
from jax.experimental.pallas import tpu_sc as plsc

T, D, E, F, K = 2048, 1024, 64, 256, 2
TR = 64
R = 8192
NB = R // TR

_NC, _NS = 2, 16
NW = _NC * _NS
TPW = T // NW


def _sc_mesh():
    return plsc.VectorSubcoreMesh(core_axis_name="c", subcore_axis_name="s")


def _wid():
    return lax.axis_index("s") * _NC + lax.axis_index("c")


_HF = TPW // 2


def _dispatch_scatter(x, se, so):
    @functools.partial(
        pl.kernel,
        out_type=jax.ShapeDtypeStruct((R, D), jnp.float32),
        mesh=_sc_mesh(),
        scratch_types=[
            pltpu.VMEM((TPW, D), jnp.float32),
            pltpu.VMEM((_HF,), jnp.int32),
            pltpu.VMEM((_HF,), jnp.int32),
            pltpu.VMEM((_HF,), jnp.int32),
            pltpu.VMEM((_HF,), jnp.int32),
            pltpu.SemaphoreType.DMA,
        ],
    )
    def k(x_hbm, se_hbm, so_hbm, xs_hbm, rows_v,
          se0_v, so0_v, se1_v, so1_v, sem):
        w = _wid()
        base = w * TPW
        pltpu.sync_copy(se_hbm.at[pl.ds(base, _HF)], se0_v)
        pltpu.sync_copy(so_hbm.at[pl.ds(base, _HF)], so0_v)
        pltpu.sync_copy(x_hbm.at[pl.ds(base, _HF)],
                        rows_v.at[pl.ds(0, _HF)])
        c0 = pltpu.async_copy(rows_v.at[pl.ds(0, _HF)],
                              xs_hbm.at[se0_v], sem)
        c1 = pltpu.async_copy(rows_v.at[pl.ds(0, _HF)],
                              xs_hbm.at[so0_v], sem)
        pltpu.sync_copy(se_hbm.at[pl.ds(base + _HF, _HF)], se1_v)
        pltpu.sync_copy(so_hbm.at[pl.ds(base + _HF, _HF)], so1_v)
        pltpu.sync_copy(x_hbm.at[pl.ds(base + _HF, _HF)],
                        rows_v.at[pl.ds(_HF, _HF)])
        c2 = pltpu.async_copy(rows_v.at[pl.ds(_HF, _HF)],
                              xs_hbm.at[se1_v], sem)
        c3 = pltpu.async_copy(rows_v.at[pl.ds(_HF, _HF)],
                              xs_hbm.at[so1_v], sem)
        c0.wait()
        c1.wait()
        c2.wait()
        c3.wait()

    return k(x, se, so)


_CH = 16
_NCH = TPW // _CH


def _combine(ys, se, so, tw0, tw1):
    @functools.partial(
        pl.kernel,
        out_type=jax.ShapeDtypeStruct((T, D), jnp.float32),
        mesh=_sc_mesh(),
        scratch_types=[
            pltpu.VMEM((2, _CH, D), jnp.float32),
            pltpu.VMEM((2, _CH, D), jnp.float32),
            pltpu.VMEM((_CH, D), jnp.float32),
            pltpu.VMEM((2, _CH), jnp.int32),
            pltpu.VMEM((2, _CH), jnp.int32),
            pltpu.VMEM((_CH, 16), jnp.float32),
            pltpu.VMEM((_CH, 16), jnp.float32),
            pltpu.SemaphoreType.DMA,
            pltpu.SemaphoreType.DMA,
        ],
    )
    def k(ys_hbm, se_hbm, so_hbm, tw0_hbm, tw1_hbm, out_hbm,
          r0_v, r1_v, ob_v, s0_v, s1_v, w0_v, w1_v, sem0, sem1):
        w = _wid()
        sems = (sem0, sem1)

        def fire(c):
            p = c % 2
            base = w * TPW + c * _CH
            pltpu.sync_copy(se_hbm.at[pl.ds(base, _CH)], s0_v.at[p])
            pltpu.sync_copy(so_hbm.at[pl.ds(base, _CH)], s1_v.at[p])
            g0 = pltpu.async_copy(ys_hbm.at[s0_v.at[p]], r0_v.at[p], sems[p])
            g1 = pltpu.async_copy(ys_hbm.at[s1_v.at[p]], r1_v.at[p], sems[p])
            return g0, g1

        hs = fire(0)
        for c in range(_NCH):
            nxt = fire(c + 1) if c + 1 < _NCH else None
            base = w * TPW + c * _CH
            pltpu.sync_copy(tw0_hbm.at[pl.ds(base, _CH)], w0_v)
            pltpu.sync_copy(tw1_hbm.at[pl.ds(base, _CH)], w1_v)
            hs[0].wait()
            hs[1].wait()
            p = c % 2
            for t in range(_CH):
                w0 = w0_v[t, :]
                w1 = w1_v[t, :]

                def body(j, _):
                    sl = pl.ds(j * 16, 16)
                    ob_v[t, sl] = (w0 * r0_v[p, t, sl]
                                   + w1 * r1_v[p, t, sl])
                    return 0

                lax.fori_loop(0, D // 16, body, 0, unroll=8)
            pltpu.sync_copy(ob_v, out_hbm.at[pl.ds(base, _CH)])
            hs = nxt

    return k(ys, se, so, tw0, tw1)


def _ffn_body(be_ref, nu_ref, xs_ref, gw_ref, uw_ref, dw_ref, ys_ref):
    i = pl.program_id(0)

    @pl.when(i < nu_ref[0])
    def _():
        xb = xs_ref[...]
        gw = gw_ref[0]
        uw = uw_ref[0]
        dw = dw_ref[0]
        cdims = (((1,), (1,)), ((), ()))
        g = jax.lax.dot_general(xb, gw, cdims,
                                preferred_element_type=jnp.float32)
        u = jax.lax.dot_general(xb, uw, cdims,
                                preferred_element_type=jnp.float32)
        h = (g * jax.nn.sigmoid(g)) * u
        ys_ref[...] = jax.lax.dot_general(h, dw, cdims,
                                          preferred_element_type=jnp.float32)


def _grouped_ffn(block_expert, num_used, xs, gpw, upw, dpw):
    grid_spec = pltpu.PrefetchScalarGridSpec(
        num_scalar_prefetch=2,
        grid=(NB,),
        in_specs=[
            pl.BlockSpec((TR, D),
                         lambda i, be, nu: (jnp.minimum(i, nu[0] - 1), 0)),
            pl.BlockSpec((1, F, D), lambda i, be, nu: (be[i], 0, 0)),
            pl.BlockSpec((1, F, D), lambda i, be, nu: (be[i], 0, 0)),
            pl.BlockSpec((1, D, F), lambda i, be, nu: (be[i], 0, 0)),
        ],
        out_specs=pl.BlockSpec(
            (TR, D), lambda i, be, nu: (jnp.minimum(i, nu[0] - 1), 0)),
    )
    return pl.pallas_call(
        _ffn_body,
        grid_spec=grid_spec,
        out_shape=jax.ShapeDtypeStruct((R, D), jnp.float32),
    )(block_expert, num_used, xs, gpw, upw, dpw)


def _top2_exact(logits):
    idx1 = jnp.argmax(logits, axis=-1).astype(jnp.int32)
    v1 = jnp.take_along_axis(logits, idx1[:, None], axis=1)[:, 0]
    masked = jnp.where(jax.nn.one_hot(idx1, E, dtype=jnp.bool_),
                       -jnp.inf, logits)
    idx2 = jnp.argmax(masked, axis=-1).astype(jnp.int32)
    v2 = jnp.take_along_axis(logits, idx2[:, None], axis=1)[:, 0]
    return jnp.stack([v1, v2], axis=1), jnp.stack([idx1, idx2], axis=1)


def kernel(hidden_states, gate_w, gate_proj_w, up_proj_w, down_proj_w):
    b, s, d = hidden_states.shape
    x = hidden_states.reshape(-1, d)

    logits = x @ gate_w.T
    topk_w_raw, topk_idx = _top2_exact(logits)
    tw = jax.nn.softmax(topk_w_raw, axis=-1)
    tw = tw / (tw.sum(axis=-1, keepdims=True) + 1e-20)

    e_flat = topk_idx.reshape(-1)
    oh = jax.nn.one_hot(e_flat, E, dtype=jnp.int16)
    counts = oh.sum(axis=0, dtype=jnp.int32)
    rank = jnp.sum((jnp.cumsum(oh, axis=0) - oh) * oh,
                   axis=1, dtype=jnp.int32)
    blocks_per_e = (counts + TR - 1) // TR
    pad_off = jnp.concatenate(
        [jnp.zeros((1,), jnp.int32),
         jnp.cumsum(blocks_per_e * TR).astype(jnp.int32)])
    slot = pad_off[e_flat] + rank
    slot2 = slot.reshape(T, K)
    se = slot2[:, 0]
    so = slot2[:, 1]
    xs = _dispatch_scatter(x, se, so)
    num_used = (pad_off[E:E + 1] // TR).astype(jnp.int32)
    blk = jnp.minimum(jnp.arange(NB, dtype=jnp.int32), num_used[0] - 1)
    block_expert = jnp.clip(
        jnp.searchsorted(pad_off, blk * TR, side='right') - 1,
        0, E - 1).astype(jnp.int32)

    ys = _grouped_ffn(block_expert, num_used, xs,
                      gate_proj_w, up_proj_w, down_proj_w)

    tw0b = jnp.broadcast_to(tw[:, 0:1], (T, 16))
    tw1b = jnp.broadcast_to(tw[:, 1:2], (T, 16))
    out = _combine(ys, se, so, tw0b, tw1b)
    return out.reshape(b, s, d)


# speedup vs baseline: 1.5046x; 1.0761x over previous
import functools

import jax
import jax.numpy as jnp
from jax import lax
from jax.experimental import pallas as pl
from jax.experimental.pallas import tpu as pltpu
from jax.experimental.pallas import tpu_sc as plsc

T, D, E, F, K = 2048, 1024, 64, 256, 2
TR = 64
R = 8192
NB = R // TR

_NC, _NS = 2, 16
NW = _NC * _NS
TPW = T // NW


def _sc_mesh():
    return plsc.VectorSubcoreMesh(core_axis_name="c", subcore_axis_name="s")


def _wid():
    return lax.axis_index("s") * _NC + lax.axis_index("c")


_HF = TPW // 2


def _dispatch_scatter(x, se, so):
    @functools.partial(
        pl.kernel,
        out_type=jax.ShapeDtypeStruct((R, D), jnp.float32),
        mesh=_sc_mesh(),
        scratch_types=[
            pltpu.VMEM((TPW, D), jnp.float32),
            pltpu.VMEM((_HF,), jnp.int32),
            pltpu.VMEM((_HF,), jnp.int32),
            pltpu.VMEM((_HF,), jnp.int32),
            pltpu.VMEM((_HF,), jnp.int32),
            pltpu.SemaphoreType.DMA,
        ],
    )
    def k(x_hbm, se_hbm, so_hbm, xs_hbm, rows_v,
          se0_v, so0_v, se1_v, so1_v, sem):
        w = _wid()
        base = w * TPW
        pltpu.sync_copy(se_hbm.at[pl.ds(base, _HF)], se0_v)
        pltpu.sync_copy(so_hbm.at[pl.ds(base, _HF)], so0_v)
        pltpu.sync_copy(x_hbm.at[pl.ds(base, _HF)],
                        rows_v.at[pl.ds(0, _HF)])
        c0 = pltpu.async_copy(rows_v.at[pl.ds(0, _HF)],
                              xs_hbm.at[se0_v], sem)
        c1 = pltpu.async_copy(rows_v.at[pl.ds(0, _HF)],
                              xs_hbm.at[so0_v], sem)
        pltpu.sync_copy(se_hbm.at[pl.ds(base + _HF, _HF)], se1_v)
        pltpu.sync_copy(so_hbm.at[pl.ds(base + _HF, _HF)], so1_v)
        pltpu.sync_copy(x_hbm.at[pl.ds(base + _HF, _HF)],
                        rows_v.at[pl.ds(_HF, _HF)])
        c2 = pltpu.async_copy(rows_v.at[pl.ds(_HF, _HF)],
                              xs_hbm.at[se1_v], sem)
        c3 = pltpu.async_copy(rows_v.at[pl.ds(_HF, _HF)],
                              xs_hbm.at[so1_v], sem)
        c0.wait()
        c1.wait()
        c2.wait()
        c3.wait()

    return k(x, se, so)


_CH = 16
_NCH = TPW // _CH


def _combine(ys, se, so, tw0, tw1):
    @functools.partial(
        pl.kernel,
        out_type=jax.ShapeDtypeStruct((T, D), jnp.float32),
        mesh=_sc_mesh(),
        scratch_types=[
            pltpu.VMEM((2, _CH, D), jnp.float32),
            pltpu.VMEM((2, _CH, D), jnp.float32),
            pltpu.VMEM((_CH, D), jnp.float32),
            pltpu.VMEM((2, _CH), jnp.int32),
            pltpu.VMEM((2, _CH), jnp.int32),
            pltpu.VMEM((_CH, 16), jnp.float32),
            pltpu.VMEM((_CH, 16), jnp.float32),
            pltpu.SemaphoreType.DMA,
            pltpu.SemaphoreType.DMA,
        ],
    )
    def k(ys_hbm, se_hbm, so_hbm, tw0_hbm, tw1_hbm, out_hbm,
          r0_v, r1_v, ob_v, s0_v, s1_v, w0_v, w1_v, sem0, sem1):
        w = _wid()
        sems = (sem0, sem1)

        def fire(c):
            p = c % 2
            base = w * TPW + c * _CH
            pltpu.sync_copy(se_hbm.at[pl.ds(base, _CH)], s0_v.at[p])
            pltpu.sync_copy(so_hbm.at[pl.ds(base, _CH)], s1_v.at[p])
            g0 = pltpu.async_copy(ys_hbm.at[s0_v.at[p]], r0_v.at[p], sems[p])
            g1 = pltpu.async_copy(ys_hbm.at[s1_v.at[p]], r1_v.at[p], sems[p])
            return g0, g1

        hs = fire(0)
        for c in range(_NCH):
            nxt = fire(c + 1) if c + 1 < _NCH else None
            base = w * TPW + c * _CH
            pltpu.sync_copy(tw0_hbm.at[pl.ds(base, _CH)], w0_v)
            pltpu.sync_copy(tw1_hbm.at[pl.ds(base, _CH)], w1_v)
            hs[0].wait()
            hs[1].wait()
            p = c % 2
            for t in range(_CH):
                w0 = w0_v[t, :]
                w1 = w1_v[t, :]

                def body(j, _):
                    sl = pl.ds(j * 16, 16)
                    ob_v[t, sl] = (w0 * r0_v[p, t, sl]
                                   + w1 * r1_v[p, t, sl])
                    return 0

                lax.fori_loop(0, D // 16, body, 0, unroll=8)
            pltpu.sync_copy(ob_v, out_hbm.at[pl.ds(base, _CH)])
            hs = nxt

    return k(ys, se, so, tw0, tw1)


def _ffn_body(be_ref, nu_ref, xs_ref, gw_ref, uw_ref, dw_ref, ys_ref):
    i = pl.program_id(0)

    @pl.when(i < nu_ref[0])
    def _():
        xb = xs_ref[...]
        gw = gw_ref[0]
        uw = uw_ref[0]
        dw = dw_ref[0]
        cdims = (((1,), (1,)), ((), ()))
        g = jax.lax.dot_general(xb, gw, cdims,
                                preferred_element_type=jnp.float32)
        u = jax.lax.dot_general(xb, uw, cdims,
                                preferred_element_type=jnp.float32)
        h = (g * jax.nn.sigmoid(g)) * u
        ys_ref[...] = jax.lax.dot_general(h, dw, cdims,
                                          preferred_element_type=jnp.float32)


def _grouped_ffn(block_expert, num_used, xs, gpw, upw, dpw):
    grid_spec = pltpu.PrefetchScalarGridSpec(
        num_scalar_prefetch=2,
        grid=(NB,),
        in_specs=[
            pl.BlockSpec((TR, D),
                         lambda i, be, nu: (jnp.minimum(i, nu[0] - 1), 0)),
            pl.BlockSpec((1, F, D), lambda i, be, nu: (be[i], 0, 0)),
            pl.BlockSpec((1, F, D), lambda i, be, nu: (be[i], 0, 0)),
            pl.BlockSpec((1, D, F), lambda i, be, nu: (be[i], 0, 0)),
        ],
        out_specs=pl.BlockSpec(
            (TR, D), lambda i, be, nu: (jnp.minimum(i, nu[0] - 1), 0)),
    )
    return pl.pallas_call(
        _ffn_body,
        grid_spec=grid_spec,
        out_shape=jax.ShapeDtypeStruct((R, D), jnp.float32),
    )(block_expert, num_used, xs, gpw, upw, dpw)


def _top2_exact(logits):
    idx1 = jnp.argmax(logits, axis=-1).astype(jnp.int32)
    v1 = jnp.take_along_axis(logits, idx1[:, None], axis=1)[:, 0]
    masked = jnp.where(jax.nn.one_hot(idx1, E, dtype=jnp.bool_),
                       -jnp.inf, logits)
    idx2 = jnp.argmax(masked, axis=-1).astype(jnp.int32)
    v2 = jnp.take_along_axis(logits, idx2[:, None], axis=1)[:, 0]
    return jnp.stack([v1, v2], axis=1), jnp.stack([idx1, idx2], axis=1)


def kernel(hidden_states, gate_w, gate_proj_w, up_proj_w, down_proj_w):
    b, s, d = hidden_states.shape
    x = hidden_states.reshape(-1, d)

    logits = x @ gate_w.T
    topk_w_raw, topk_idx = _top2_exact(logits)
    tw = jax.nn.softmax(topk_w_raw, axis=-1)
    tw = tw / (tw.sum(axis=-1, keepdims=True) + 1e-20)

    e_flat = topk_idx.reshape(-1)
    oh = jax.nn.one_hot(e_flat, E, dtype=jnp.float32)
    ohb = oh.reshape(32, (T * K) // 32, E)
    ltri = jnp.tril(jnp.ones(((T * K) // 32,) * 2, jnp.float32), -1)
    intra = jnp.einsum('ij,bjk->bik', ltri, ohb,
                       preferred_element_type=jnp.float32)
    bsum = ohb.sum(axis=1)
    offs = jnp.cumsum(bsum, axis=0) - bsum
    rank_mat = (intra + offs[:, None, :]).reshape(T * K, E)
    counts = bsum.sum(axis=0).astype(jnp.int32)
    rank = jnp.sum(rank_mat * oh, axis=1).astype(jnp.int32)
    blocks_per_e = (counts + TR - 1) // TR
    pad_off = jnp.concatenate(
        [jnp.zeros((1,), jnp.int32),
         jnp.cumsum(blocks_per_e * TR).astype(jnp.int32)])
    slot = pad_off[e_flat] + rank
    slot2 = slot.reshape(T, K)
    se = slot2[:, 0]
    so = slot2[:, 1]
    xs = _dispatch_scatter(x, se, so)
    num_used = (pad_off[E:E + 1] // TR).astype(jnp.int32)
    blk = jnp.minimum(jnp.arange(NB, dtype=jnp.int32), num_used[0] - 1)
    block_expert = jnp.clip(
        jnp.searchsorted(pad_off, blk * TR, side='right') - 1,
        0, E - 1).astype(jnp.int32)

    ys = _grouped_ffn(block_expert, num_used, xs,
                      gate_proj_w, up_proj_w, down_proj_w)

    tw0b = jnp.broadcast_to(tw[:, 0:1], (T, 16))
    tw1b = jnp.broadcast_to(tw[:, 1:2], (T, 16))
    out = _combine(ys, se, so, tw0b, tw1b)
    return out.reshape(b, s, d)
